# Initial kernel scaffold; baseline (speedup 1.0000x reference)
import jax
import jax.numpy as jnp
from jax.experimental import pallas as pl


def kernel(x_local, x_global, edge_attr, edge_index, params):
    raise NotImplementedError("write your pallas kernel here")


# speedup vs baseline: 24.0331x; 24.0331x over previous
import functools

import jax
import jax.numpy as jnp
from jax import lax
from jax.experimental import pallas as pl
from jax.experimental.pallas import tpu as pltpu
from jax.experimental.pallas import tpu_sc as plsc

N = 10000
E = 320000
D = 128
H = 64
A = 32
SIG2 = 900.0

NC = 2
NS = 16
NW = NC * NS
EPT = E // NW
CH = 80
NCH = EPT // CH
NT = 10
SPAN = N // NT
ZR = 200
ZS = 1000
L = 16

_mesh = plsc.VectorSubcoreMesh(core_axis_name="c", subcore_axis_name="s")


def _ew_body(ea_ref, w1_ref, b1_ref, w2_ref, b2_ref, out_ref):
    h = jnp.dot(ea_ref[...], w1_ref[...], preferred_element_type=jnp.float32)
    h = h + b1_ref[...]
    h = jnp.where(h > 0, h, jnp.exp(h) - 1.0)
    ind = jnp.dot(h, w2_ref[...], preferred_element_type=jnp.float32)
    ind = ind + b2_ref[...]
    out_ref[...] = jnp.exp(-(ind * ind) / SIG2)


def _edge_weights(edge_attr, p):
    BE = 4000
    return pl.pallas_call(
        _ew_body,
        grid=(E // BE,),
        in_specs=[
            pl.BlockSpec((BE, 10), lambda i: (i, 0)),
            pl.BlockSpec((10, 10), lambda i: (0, 0)),
            pl.BlockSpec((1, 10), lambda i: (0, 0)),
            pl.BlockSpec((10, 1), lambda i: (0, 0)),
            pl.BlockSpec((1, 1), lambda i: (0, 0)),
        ],
        out_specs=pl.BlockSpec((BE, 1), lambda i: (i, 0)),
        out_shape=jax.ShapeDtypeStruct((E, 1), jnp.float32),
    )(edge_attr, p["ew1_w"], p["ew1_b"].reshape(1, 10),
      p["ew2_w"].reshape(10, 1), p["ew2_b"].reshape(1, 1))


def _prep_body(xl_ref, xg_ref, w0_ref, b0_ref, as_ref, ad_ref, wg_ref,
               zt_ref, asrc_ref, adst_ref, zz_ref):
    zt = jnp.dot(xl_ref[...], w0_ref[...], preferred_element_type=jnp.float32)
    zt = zt + b0_ref[...]
    zt_ref[...] = zt
    asrc_ref[...] = jnp.dot(zt, as_ref[...], preferred_element_type=jnp.float32)
    adst_ref[...] = jnp.dot(zt, ad_ref[...], preferred_element_type=jnp.float32)
    zz_ref[...] = jnp.dot(xg_ref[...], wg_ref[...],
                          preferred_element_type=jnp.float32)


def _node_prep(x_local, x_global, p):
    BN = 2000
    g0 = p["gat0"]
    return pl.pallas_call(
        _prep_body,
        grid=(N // BN,),
        in_specs=[
            pl.BlockSpec((BN, D), lambda i: (i, 0)),
            pl.BlockSpec((BN, D), lambda i: (i, 0)),
            pl.BlockSpec((D, H), lambda i: (0, 0)),
            pl.BlockSpec((1, H), lambda i: (0, 0)),
            pl.BlockSpec((H, 1), lambda i: (0, 0)),
            pl.BlockSpec((H, 1), lambda i: (0, 0)),
            pl.BlockSpec((D, H), lambda i: (0, 0)),
        ],
        out_specs=[
            pl.BlockSpec((BN, H), lambda i: (i, 0)),
            pl.BlockSpec((BN, 1), lambda i: (i, 0)),
            pl.BlockSpec((BN, 1), lambda i: (i, 0)),
            pl.BlockSpec((BN, H), lambda i: (i, 0)),
        ],
        out_shape=[
            jax.ShapeDtypeStruct((N, H), jnp.float32),
            jax.ShapeDtypeStruct((N, 1), jnp.float32),
            jax.ShapeDtypeStruct((N, 1), jnp.float32),
            jax.ShapeDtypeStruct((N, H), jnp.float32),
        ],
    )(x_local, x_global, g0["W"], g0["b"].reshape(1, H),
      g0["a_src"].reshape(H, 1), g0["a_dst"].reshape(H, 1), p["gcn0"]["W"])


@functools.partial(
    pl.kernel,
    out_type=[
        jax.ShapeDtypeStruct((NC, NT, SPAN), jnp.float32),
        jax.ShapeDtypeStruct((NC, NT, SPAN), jnp.float32),
        jax.ShapeDtypeStruct((NC, NT, SPAN), jnp.float32),
        jax.ShapeDtypeStruct((NC, NT, SPAN, H), jnp.float32),
    ],
    mesh=_mesh,
    compiler_params=pltpu.CompilerParams(needs_layout_passes=False, use_tc_tiling_on_sc=False),
    scratch_types=[
        pltpu.VMEM((N,), jnp.float32),
        pltpu.VMEM((N,), jnp.float32),
        pltpu.VMEM((EPT,), jnp.int32),
        pltpu.VMEM((EPT,), jnp.int32),
        pltpu.VMEM((EPT,), jnp.float32),
        pltpu.VMEM((CH,), jnp.int32),
        pltpu.VMEM((CH,), jnp.int32),
        pltpu.VMEM((CH,), jnp.float32),
        pltpu.VMEM((CH,), jnp.float32),
        pltpu.VMEM((CH, H), jnp.float32),
        pltpu.VMEM((ZR, H), jnp.float32),
        pltpu.VMEM((ZS,), jnp.float32),
        pltpu.VMEM_SHARED((N,), jnp.float32),
        pltpu.VMEM_SHARED((N,), jnp.float32),
        pltpu.VMEM_SHARED((N,), jnp.float32),
        pltpu.VMEM_SHARED((N, H), jnp.float32),
        pltpu.SemaphoreType.DMA,
    ],
)
def _sc_gat0(src_hbm, dst_hbm, ew_hbm, asrc_hbm, adst_hbm, zt_hbm,
             den_out, degd_out, degs_out, acc_out,
             asrc_t, adst_t, src_v, dst_v, ew_v,
             srcbuf, dstbuf, ewbuf, exbuf, rows, zrows, zsc,
             den_sh, degd_sh, degs_sh, acc_sh, sem):
    cid = lax.axis_index("c")
    sid = lax.axis_index("s")
    wid = cid * NS + sid
    ebase = wid * EPT

    pltpu.sync_copy(src_hbm.at[pl.ds(ebase, EPT)], src_v)
    pltpu.sync_copy(dst_hbm.at[pl.ds(ebase, EPT)], dst_v)
    pltpu.sync_copy(ew_hbm.at[pl.ds(ebase, EPT)], ew_v)
    pltpu.sync_copy(asrc_hbm, asrc_t)
    pltpu.sync_copy(adst_hbm, adst_t)

    zv = jnp.zeros((L,), jnp.float32)
    for i in range(ZR):
        for q in range(H // L):
            zrows[i, pl.ds(L * q, L)] = zv
    for i in range(ZS // L):
        zsc[pl.ds(L * i, L)] = zv

    @pl.when(sid < NT)
    def _zero_accs():
        for k in range(SPAN // ZR):
            pltpu.sync_copy(zrows, acc_sh.at[pl.ds(sid * SPAN + k * ZR, ZR), :])
        pltpu.sync_copy(zsc, den_sh.at[pl.ds(sid * SPAN, ZS)])
        pltpu.sync_copy(zsc, degd_sh.at[pl.ds(sid * SPAN, ZS)])
        pltpu.sync_copy(zsc, degs_sh.at[pl.ds(sid * SPAN, ZS)])

    plsc.subcore_barrier()

    def chunk(j, carry):
        b = j * CH
        for t in range(CH // L):
            srcbuf[pl.ds(L * t, L)] = src_v[pl.ds(b + L * t, L)]
            dstbuf[pl.ds(L * t, L)] = dst_v[pl.ds(b + L * t, L)]
            ewbuf[pl.ds(L * t, L)] = ew_v[pl.ds(b + L * t, L)]
        pltpu.async_copy(zt_hbm.at[srcbuf], rows, sem).wait()
        for t in range(CH // L):
            si = srcbuf[pl.ds(L * t, L)]
            di = dstbuf[pl.ds(L * t, L)]
            lg = plsc.load_gather(asrc_t, [si]) + plsc.load_gather(adst_t, [di])
            lg = jnp.where(lg > 0, lg, 0.2 * lg)
            exbuf[pl.ds(L * t, L)] = jnp.exp(lg) * ewbuf[pl.ds(L * t, L)]
        for t in range(CH // L):
            ex16 = exbuf[pl.ds(L * t, L)]
            for k in range(L):
                i = L * t + k
                s = jnp.full((L,), ex16[k], jnp.float32)
                for q in range(H // L):
                    rows[i, pl.ds(L * q, L)] = rows[i, pl.ds(L * q, L)] * s
        pltpu.sync_copy(rows, acc_sh.at[dstbuf], add=True)
        pltpu.sync_copy(exbuf, den_sh.at[dstbuf], add=True)
        pltpu.sync_copy(ewbuf, degd_sh.at[dstbuf], add=True)
        pltpu.sync_copy(ewbuf, degs_sh.at[srcbuf], add=True)
        return carry

    lax.fori_loop(0, NCH, chunk, 0)
    plsc.subcore_barrier()

    @pl.when(sid < NT)
    def _copy_out():
        pltpu.sync_copy(acc_sh.at[pl.ds(sid * SPAN, SPAN), :],
                        acc_out.at[cid, sid])
        pltpu.sync_copy(den_sh.at[pl.ds(sid * SPAN, SPAN)],
                        den_out.at[cid, sid])
        pltpu.sync_copy(degd_sh.at[pl.ds(sid * SPAN, SPAN)],
                        degd_out.at[cid, sid])
        pltpu.sync_copy(degs_sh.at[pl.ds(sid * SPAN, SPAN)],
                        degs_out.at[cid, sid])


def _mid_body(a0_ref, a1_ref, d0_ref, d1_ref, dd0_ref, dd1_ref,
              ds0_ref, ds1_ref, bng_ref, bnb_ref, w1_ref, b1_ref,
              as1_ref, ad1_ref,
              zt1_ref, asrc1_ref, adst1_ref, rsds_ref, rsdd_ref):
    den = d0_ref[...] + d1_ref[...] + 1e-16
    hl1 = (a0_ref[...] + a1_ref[...]) / den
    hl1 = jnp.maximum(hl1 * bng_ref[...] + bnb_ref[...], 0.0)
    zt1 = jnp.dot(hl1, w1_ref[...], preferred_element_type=jnp.float32)
    zt1 = zt1 + b1_ref[...]
    zt1_ref[...] = zt1
    asrc1_ref[...] = jnp.dot(zt1, as1_ref[...],
                             preferred_element_type=jnp.float32)
    adst1_ref[...] = jnp.dot(zt1, ad1_ref[...],
                             preferred_element_type=jnp.float32)
    rsds_ref[...] = lax.rsqrt(ds0_ref[...] + ds1_ref[...] + 1.0)
    rsdd_ref[...] = lax.rsqrt(dd0_ref[...] + dd1_ref[...] + 1.0)


def _mid(acc0, den0, degd, degs, p):
    BN = 2000
    g1 = p["gat1"]
    g0 = p["gat0"]
    nvec = lambda v: v.reshape(N, 1)
    return pl.pallas_call(
        _mid_body,
        grid=(N // BN,),
        in_specs=[
            pl.BlockSpec((BN, H), lambda i: (i, 0)),
            pl.BlockSpec((BN, H), lambda i: (i, 0)),
            pl.BlockSpec((BN, 1), lambda i: (i, 0)),
            pl.BlockSpec((BN, 1), lambda i: (i, 0)),
            pl.BlockSpec((BN, 1), lambda i: (i, 0)),
            pl.BlockSpec((BN, 1), lambda i: (i, 0)),
            pl.BlockSpec((BN, 1), lambda i: (i, 0)),
            pl.BlockSpec((BN, 1), lambda i: (i, 0)),
            pl.BlockSpec((1, H), lambda i: (0, 0)),
            pl.BlockSpec((1, H), lambda i: (0, 0)),
            pl.BlockSpec((H, H), lambda i: (0, 0)),
            pl.BlockSpec((1, H), lambda i: (0, 0)),
            pl.BlockSpec((H, 1), lambda i: (0, 0)),
            pl.BlockSpec((H, 1), lambda i: (0, 0)),
        ],
        out_specs=[
            pl.BlockSpec((BN, H), lambda i: (i, 0)),
            pl.BlockSpec((BN, 1), lambda i: (i, 0)),
            pl.BlockSpec((BN, 1), lambda i: (i, 0)),
            pl.BlockSpec((BN, 1), lambda i: (i, 0)),
            pl.BlockSpec((BN, 1), lambda i: (i, 0)),
        ],
        out_shape=[
            jax.ShapeDtypeStruct((N, H), jnp.float32),
            jax.ShapeDtypeStruct((N, 1), jnp.float32),
            jax.ShapeDtypeStruct((N, 1), jnp.float32),
            jax.ShapeDtypeStruct((N, 1), jnp.float32),
            jax.ShapeDtypeStruct((N, 1), jnp.float32),
        ],
    )(acc0[0], acc0[1], nvec(den0[0]), nvec(den0[1]),
      nvec(degd[0]), nvec(degd[1]), nvec(degs[0]), nvec(degs[1]),
      g0["bn_g"].reshape(1, H), g0["bn_b"].reshape(1, H),
      g1["W"], g1["b"].reshape(1, H),
      g1["a_src"].reshape(H, 1), g1["a_dst"].reshape(H, 1))


@functools.partial(
    pl.kernel,
    out_type=[
        jax.ShapeDtypeStruct((NC, NT, SPAN), jnp.float32),
        jax.ShapeDtypeStruct((NC, NT, SPAN, H), jnp.float32),
    ],
    mesh=_mesh,
    compiler_params=pltpu.CompilerParams(needs_layout_passes=False, use_tc_tiling_on_sc=False),
    scratch_types=[
        pltpu.VMEM((N,), jnp.float32),
        pltpu.VMEM((N,), jnp.float32),
        pltpu.VMEM((EPT,), jnp.int32),
        pltpu.VMEM((EPT,), jnp.int32),
        pltpu.VMEM((EPT,), jnp.float32),
        pltpu.VMEM((CH,), jnp.int32),
        pltpu.VMEM((CH,), jnp.int32),
        pltpu.VMEM((CH,), jnp.float32),
        pltpu.VMEM((CH,), jnp.float32),
        pltpu.VMEM((CH, H), jnp.float32),
        pltpu.VMEM((ZR, H), jnp.float32),
        pltpu.VMEM((ZS,), jnp.float32),
        pltpu.VMEM_SHARED((N,), jnp.float32),
        pltpu.VMEM_SHARED((N, H), jnp.float32),
        pltpu.SemaphoreType.DMA,
    ],
)
def _sc_gat1(src_hbm, dst_hbm, ew_hbm, asrc_hbm, adst_hbm, zt1_hbm,
             den_out, acc1_out,
             asrc_t, adst_t, src_v, dst_v, ew_v,
             srcbuf, dstbuf, ewbuf, exbuf, rows,
             zrows, zsc, den_sh, acc1_sh, sem):
    cid = lax.axis_index("c")
    sid = lax.axis_index("s")
    wid = cid * NS + sid
    ebase = wid * EPT

    pltpu.sync_copy(src_hbm.at[pl.ds(ebase, EPT)], src_v)
    pltpu.sync_copy(dst_hbm.at[pl.ds(ebase, EPT)], dst_v)
    pltpu.sync_copy(ew_hbm.at[pl.ds(ebase, EPT)], ew_v)
    pltpu.sync_copy(asrc_hbm, asrc_t)
    pltpu.sync_copy(adst_hbm, adst_t)

    zv = jnp.zeros((L,), jnp.float32)
    for i in range(ZR):
        for q in range(H // L):
            zrows[i, pl.ds(L * q, L)] = zv
    for i in range(ZS // L):
        zsc[pl.ds(L * i, L)] = zv

    @pl.when(sid < NT)
    def _zero_accs():
        for k in range(SPAN // ZR):
            pltpu.sync_copy(zrows, acc1_sh.at[pl.ds(sid * SPAN + k * ZR, ZR), :])
        pltpu.sync_copy(zsc, den_sh.at[pl.ds(sid * SPAN, ZS)])

    plsc.subcore_barrier()

    def chunk(j, carry):
        b = j * CH
        for t in range(CH // L):
            srcbuf[pl.ds(L * t, L)] = src_v[pl.ds(b + L * t, L)]
            dstbuf[pl.ds(L * t, L)] = dst_v[pl.ds(b + L * t, L)]
            ewbuf[pl.ds(L * t, L)] = ew_v[pl.ds(b + L * t, L)]
        cp1 = pltpu.async_copy(zt1_hbm.at[srcbuf], rows, sem)
        for t in range(CH // L):
            si = srcbuf[pl.ds(L * t, L)]
            di = dstbuf[pl.ds(L * t, L)]
            lg = plsc.load_gather(asrc_t, [si]) + plsc.load_gather(adst_t, [di])
            lg = jnp.where(lg > 0, lg, 0.2 * lg)
            exbuf[pl.ds(L * t, L)] = jnp.exp(lg) * ewbuf[pl.ds(L * t, L)]
        cp1.wait()
        for t in range(CH // L):
            ex16 = exbuf[pl.ds(L * t, L)]
            for k in range(L):
                i = L * t + k
                s = jnp.full((L,), ex16[k], jnp.float32)
                for q in range(H // L):
                    rows[i, pl.ds(L * q, L)] = rows[i, pl.ds(L * q, L)] * s
        pltpu.sync_copy(rows, acc1_sh.at[dstbuf], add=True)
        pltpu.sync_copy(exbuf, den_sh.at[dstbuf], add=True)
        return carry

    lax.fori_loop(0, NCH, chunk, 0)
    plsc.subcore_barrier()

    @pl.when(sid < NT)
    def _copy_out():
        pltpu.sync_copy(acc1_sh.at[pl.ds(sid * SPAN, SPAN), :],
                        acc1_out.at[cid, sid])
        pltpu.sync_copy(den_sh.at[pl.ds(sid * SPAN, SPAN)],
                        den_out.at[cid, sid])


@functools.partial(
    pl.kernel,
    out_type=[
        jax.ShapeDtypeStruct((NC, NT, SPAN, H), jnp.float32),
    ],
    mesh=_mesh,
    compiler_params=pltpu.CompilerParams(needs_layout_passes=False, use_tc_tiling_on_sc=False),
    scratch_types=[
        pltpu.VMEM((N,), jnp.float32),
        pltpu.VMEM((N,), jnp.float32),
        pltpu.VMEM((EPT,), jnp.int32),
        pltpu.VMEM((EPT,), jnp.int32),
        pltpu.VMEM((EPT,), jnp.float32),
        pltpu.VMEM((CH,), jnp.int32),
        pltpu.VMEM((CH,), jnp.int32),
        pltpu.VMEM((CH,), jnp.float32),
        pltpu.VMEM((CH, H), jnp.float32),
        pltpu.VMEM((ZR, H), jnp.float32),
        pltpu.VMEM_SHARED((N, H), jnp.float32),
        pltpu.SemaphoreType.DMA,
    ],
)
def _sc_gcn(src_hbm, dst_hbm, ew_hbm, rsds_hbm, rsdd_hbm, zz_hbm,
            accg_out,
            rsds_t, rsdd_t, src_v, dst_v, ew_v,
            srcbuf, dstbuf, nrbuf, rowsg, zrows, accg_sh, sem):
    cid = lax.axis_index("c")
    sid = lax.axis_index("s")
    wid = cid * NS + sid
    ebase = wid * EPT

    pltpu.sync_copy(src_hbm.at[pl.ds(ebase, EPT)], src_v)
    pltpu.sync_copy(dst_hbm.at[pl.ds(ebase, EPT)], dst_v)
    pltpu.sync_copy(ew_hbm.at[pl.ds(ebase, EPT)], ew_v)
    pltpu.sync_copy(rsds_hbm, rsds_t)
    pltpu.sync_copy(rsdd_hbm, rsdd_t)

    zv = jnp.zeros((L,), jnp.float32)
    for i in range(ZR):
        for q in range(H // L):
            zrows[i, pl.ds(L * q, L)] = zv

    @pl.when(sid < NT)
    def _zero_accs():
        for k in range(SPAN // ZR):
            pltpu.sync_copy(zrows, accg_sh.at[pl.ds(sid * SPAN + k * ZR, ZR), :])

    plsc.subcore_barrier()

    def chunk(j, carry):
        b = j * CH
        for t in range(CH // L):
            srcbuf[pl.ds(L * t, L)] = src_v[pl.ds(b + L * t, L)]
            dstbuf[pl.ds(L * t, L)] = dst_v[pl.ds(b + L * t, L)]
        cp = pltpu.async_copy(zz_hbm.at[srcbuf], rowsg, sem)
        for t in range(CH // L):
            si = srcbuf[pl.ds(L * t, L)]
            di = dstbuf[pl.ds(L * t, L)]
            nr = plsc.load_gather(rsds_t, [si]) * plsc.load_gather(rsdd_t, [di])
            nrbuf[pl.ds(L * t, L)] = nr * ew_v[pl.ds(b + L * t, L)]
        cp.wait()
        for t in range(CH // L):
            nr16 = nrbuf[pl.ds(L * t, L)]
            for k in range(L):
                i = L * t + k
                g = jnp.full((L,), nr16[k], jnp.float32)
                for q in range(H // L):
                    rowsg[i, pl.ds(L * q, L)] = rowsg[i, pl.ds(L * q, L)] * g
        pltpu.sync_copy(rowsg, accg_sh.at[dstbuf], add=True)
        return carry

    lax.fori_loop(0, NCH, chunk, 0)
    plsc.subcore_barrier()

    @pl.when(sid < NT)
    def _copy_out():
        pltpu.sync_copy(accg_sh.at[pl.ds(sid * SPAN, SPAN), :],
                        accg_out.at[cid, sid])


def _head_body(a0_ref, a1_ref, d0_ref, d1_ref, g0_ref, g1_ref,
               bng_ref, bnb_ref, la1w_ref, la1b_ref, la2w_ref, la2b_ref,
               gcnb_ref, lng_ref, lnb_ref,
               ga1w_ref, ga1b_ref, ga2w_ref, ga2b_ref,
               gwl_ref, gwg_ref, gb_ref, f1l_ref, f1g_ref, f1b_ref,
               f2w_ref, f2b_ref, ow_ref, ob_ref,
               pred_ref, fused_ref):
    f32 = jnp.float32
    den = d0_ref[...] + d1_ref[...] + 1e-16
    hl2 = (a0_ref[...] + a1_ref[...]) / den
    hl2 = jnp.maximum(hl2 * bng_ref[...] + bnb_ref[...], 0.0)
    t = jnp.maximum(jnp.dot(hl2, la1w_ref[...], preferred_element_type=f32)
                    + la1b_ref[...], 0.0)
    hl = hl2 + jnp.dot(t, la2w_ref[...], preferred_element_type=f32) \
        + la2b_ref[...]

    hg = g0_ref[...] + g1_ref[...] + gcnb_ref[...]
    mu = jnp.mean(hg, axis=-1, keepdims=True)
    var = jnp.mean((hg - mu) * (hg - mu), axis=-1, keepdims=True)
    hg = (hg - mu) * lax.rsqrt(var + 1e-5) * lng_ref[...] + lnb_ref[...]
    hg = jnp.maximum(hg, 0.0)
    t = jnp.maximum(jnp.dot(hg, ga1w_ref[...], preferred_element_type=f32)
                    + ga1b_ref[...], 0.0)
    hg = hg + jnp.dot(t, ga2w_ref[...], preferred_element_type=f32) \
        + ga2b_ref[...]

    glog = jnp.dot(hl, gwl_ref[...], preferred_element_type=f32) \
        + jnp.dot(hg, gwg_ref[...], preferred_element_type=f32) + gb_ref[...]
    gate = 1.0 / (1.0 + jnp.exp(-glog))
    t = jnp.maximum(jnp.dot(hl, f1l_ref[...], preferred_element_type=f32)
                    + jnp.dot(hg, f1g_ref[...], preferred_element_type=f32)
                    + f1b_ref[...], 0.0)
    adj = jnp.dot(t, f2w_ref[...], preferred_element_type=f32) + f2b_ref[...]
    fused = gate * hl + (1.0 - gate) * hg + adj
    fused_ref[...] = fused
    pred_ref[...] = jnp.dot(fused, ow_ref[...], preferred_element_type=f32) \
        + ob_ref[...]


def _head(acc1, den1, accg, p):
    BN = 2000
    g1 = p["gat1"]
    gcn = p["gcn0"]
    nvec = lambda v: v.reshape(N, 1)
    row = lambda v: v.reshape(1, -1)
    bspec = [
        pl.BlockSpec((BN, H), lambda i: (i, 0)),
        pl.BlockSpec((BN, H), lambda i: (i, 0)),
        pl.BlockSpec((BN, 1), lambda i: (i, 0)),
        pl.BlockSpec((BN, 1), lambda i: (i, 0)),
        pl.BlockSpec((BN, H), lambda i: (i, 0)),
        pl.BlockSpec((BN, H), lambda i: (i, 0)),
        pl.BlockSpec((1, H), lambda i: (0, 0)),
        pl.BlockSpec((1, H), lambda i: (0, 0)),
        pl.BlockSpec((H, A), lambda i: (0, 0)),
        pl.BlockSpec((1, A), lambda i: (0, 0)),
        pl.BlockSpec((A, H), lambda i: (0, 0)),
        pl.BlockSpec((1, H), lambda i: (0, 0)),
        pl.BlockSpec((1, H), lambda i: (0, 0)),
        pl.BlockSpec((1, H), lambda i: (0, 0)),
        pl.BlockSpec((1, H), lambda i: (0, 0)),
        pl.BlockSpec((H, A), lambda i: (0, 0)),
        pl.BlockSpec((1, A), lambda i: (0, 0)),
        pl.BlockSpec((A, H), lambda i: (0, 0)),
        pl.BlockSpec((1, H), lambda i: (0, 0)),
        pl.BlockSpec((H, H), lambda i: (0, 0)),
        pl.BlockSpec((H, H), lambda i: (0, 0)),
        pl.BlockSpec((1, H), lambda i: (0, 0)),
        pl.BlockSpec((H, H), lambda i: (0, 0)),
        pl.BlockSpec((H, H), lambda i: (0, 0)),
        pl.BlockSpec((1, H), lambda i: (0, 0)),
        pl.BlockSpec((H, H), lambda i: (0, 0)),
        pl.BlockSpec((1, H), lambda i: (0, 0)),
        pl.BlockSpec((H, 1), lambda i: (0, 0)),
        pl.BlockSpec((1, 1), lambda i: (0, 0)),
    ]
    return pl.pallas_call(
        _head_body,
        grid=(N // BN,),
        in_specs=bspec,
        out_specs=[
            pl.BlockSpec((BN, 1), lambda i: (i, 0)),
            pl.BlockSpec((BN, H), lambda i: (i, 0)),
        ],
        out_shape=[
            jax.ShapeDtypeStruct((N, 1), jnp.float32),
            jax.ShapeDtypeStruct((N, H), jnp.float32),
        ],
    )(acc1[0], acc1[1], nvec(den1[0]), nvec(den1[1]), accg[0], accg[1],
      row(g1["bn_g"]), row(g1["bn_b"]),
      p["la1_w"], row(p["la1_b"]), p["la2_w"], row(p["la2_b"]),
      row(gcn["b"]), row(gcn["ln_g"]), row(gcn["ln_b"]),
      p["ga1_w"], row(p["ga1_b"]), p["ga2_w"], row(p["ga2_b"]),
      p["gate_w"][:H], p["gate_w"][H:], row(p["gate_b"]),
      p["fa1_w"][:H], p["fa1_w"][H:], row(p["fa1_b"]),
      p["fa2_w"], row(p["fa2_b"]), p["out_w"], p["out_b"].reshape(1, 1))


def kernel(x_local, x_global, edge_attr, edge_index, params):
    src = edge_index[0]
    dst = edge_index[1]
    ew = _edge_weights(edge_attr, params).reshape(E)
    zt0, asrc0, adst0, zz = _node_prep(x_local, x_global, params)
    den0, degd, degs, acc0 = _sc_gat0(
        src, dst, ew, asrc0.reshape(N), adst0.reshape(N), zt0)
    den0 = den0.reshape(NC, N)
    degd = degd.reshape(NC, N)
    degs = degs.reshape(NC, N)
    acc0 = acc0.reshape(NC, N, H)
    zt1, asrc1, adst1, rsds, rsdd = _mid(acc0, den0, degd, degs, params)
    den1, acc1 = _sc_gat1(
        src, dst, ew, asrc1.reshape(N), adst1.reshape(N), zt1)
    (accg,) = _sc_gcn(
        src, dst, ew, rsds.reshape(N), rsdd.reshape(N), zz)
    den1 = den1.reshape(NC, N)
    acc1 = acc1.reshape(NC, N, H)
    accg = accg.reshape(NC, N, H)
    pred, fused = _head(acc1, den1, accg, params)
    return pred.reshape(N), fused


# speedup vs baseline: 35.7163x; 1.4861x over previous
import functools

import jax
import jax.numpy as jnp
from jax import lax
from jax.experimental import pallas as pl
from jax.experimental.pallas import tpu as pltpu
from jax.experimental.pallas import tpu_sc as plsc

N = 10000
E = 320000
D = 128
H = 64
A = 32
SIG2 = 900.0

NC = 2
NS = 16
NW = NC * NS
EPT = E // NW
CH = 80
NCH = EPT // CH
NT = 10
SPAN = N // NT
ZR = 200
ZS = 1000
L = 16

_mesh = plsc.VectorSubcoreMesh(core_axis_name="c", subcore_axis_name="s")


def _ew_body(ea_ref, w1_ref, b1_ref, w2_ref, b2_ref, out_ref):
    h = lax.dot_general(w1_ref[...], ea_ref[...], (((0,), (0,)), ((), ())),
                        preferred_element_type=jnp.float32)
    h = h + b1_ref[...]
    h = jnp.where(h > 0, h, jnp.exp(h) - 1.0)
    ind = jnp.dot(w2_ref[...], h, preferred_element_type=jnp.float32)
    ind = ind + b2_ref[...]
    BE = ind.shape[-1]
    out_ref[...] = jnp.exp(-(ind * ind) / SIG2).reshape(1, 1, BE)


def _edge_weights(ea_t, p):
    BE = 32000
    return pl.pallas_call(
        _ew_body,
        grid=(E // BE,),
        in_specs=[
            pl.BlockSpec((10, BE), lambda i: (0, i)),
            pl.BlockSpec((10, 10), lambda i: (0, 0)),
            pl.BlockSpec((10, 1), lambda i: (0, 0)),
            pl.BlockSpec((1, 10), lambda i: (0, 0)),
            pl.BlockSpec((1, 1), lambda i: (0, 0)),
        ],
        out_specs=pl.BlockSpec((1, 1, BE), lambda i: (i, 0, 0)),
        out_shape=jax.ShapeDtypeStruct((E // BE, 1, BE), jnp.float32),
    )(ea_t, p["ew1_w"], p["ew1_b"].reshape(10, 1),
      p["ew2_w"].reshape(1, 10), p["ew2_b"].reshape(1, 1))


def _prep_body(xl_ref, xg_ref, w0_ref, b0_ref, as_ref, ad_ref, wg_ref,
               zt_ref, asrc_ref, adst_ref, zz_ref):
    f32 = jnp.float32
    zt = jnp.dot(xl_ref[...], w0_ref[...], preferred_element_type=f32)
    zt = zt + b0_ref[...]
    zt_ref[...] = zt
    BN = zt.shape[0]
    asrc_ref[...] = lax.dot_general(
        as_ref[...], zt, (((1,), (1,)), ((), ())),
        preferred_element_type=f32).reshape(1, 1, BN)
    adst_ref[...] = lax.dot_general(
        ad_ref[...], zt, (((1,), (1,)), ((), ())),
        preferred_element_type=f32).reshape(1, 1, BN)
    zz_ref[...] = jnp.dot(xg_ref[...], wg_ref[...],
                          preferred_element_type=f32)


def _node_prep(x_local, x_global, p):
    BN = 2000
    NB = N // BN
    g0 = p["gat0"]
    return pl.pallas_call(
        _prep_body,
        grid=(NB,),
        in_specs=[
            pl.BlockSpec((BN, D), lambda i: (i, 0)),
            pl.BlockSpec((BN, D), lambda i: (i, 0)),
            pl.BlockSpec((D, H), lambda i: (0, 0)),
            pl.BlockSpec((1, H), lambda i: (0, 0)),
            pl.BlockSpec((1, H), lambda i: (0, 0)),
            pl.BlockSpec((1, H), lambda i: (0, 0)),
            pl.BlockSpec((D, H), lambda i: (0, 0)),
        ],
        out_specs=[
            pl.BlockSpec((BN, H), lambda i: (i, 0)),
            pl.BlockSpec((1, 1, BN), lambda i: (i, 0, 0)),
            pl.BlockSpec((1, 1, BN), lambda i: (i, 0, 0)),
            pl.BlockSpec((BN, H), lambda i: (i, 0)),
        ],
        out_shape=[
            jax.ShapeDtypeStruct((N, H), jnp.float32),
            jax.ShapeDtypeStruct((NB, 1, BN), jnp.float32),
            jax.ShapeDtypeStruct((NB, 1, BN), jnp.float32),
            jax.ShapeDtypeStruct((N, H), jnp.float32),
        ],
    )(x_local, x_global, g0["W"], g0["b"].reshape(1, H),
      g0["a_src"].reshape(1, H), g0["a_dst"].reshape(1, H), p["gcn0"]["W"])


@functools.partial(
    pl.kernel,
    out_type=[
        jax.ShapeDtypeStruct((NC, NT, SPAN), jnp.float32),
        jax.ShapeDtypeStruct((NC, NT, SPAN), jnp.float32),
        jax.ShapeDtypeStruct((NC, NT, SPAN), jnp.float32),
        jax.ShapeDtypeStruct((NC, N, H), jnp.float32),
    ],
    mesh=_mesh,
    compiler_params=pltpu.CompilerParams(needs_layout_passes=False, use_tc_tiling_on_sc=False),
    scratch_types=[
        pltpu.VMEM((N,), jnp.float32),
        pltpu.VMEM((N,), jnp.float32),
        pltpu.VMEM((EPT,), jnp.int32),
        pltpu.VMEM((EPT,), jnp.int32),
        pltpu.VMEM((EPT,), jnp.float32),
        pltpu.VMEM((CH,), jnp.int32),
        pltpu.VMEM((CH,), jnp.int32),
        pltpu.VMEM((CH,), jnp.float32),
        pltpu.VMEM((CH,), jnp.float32),
        pltpu.VMEM((CH, H), jnp.float32),
        pltpu.VMEM((ZR, H), jnp.float32),
        pltpu.VMEM((ZS,), jnp.float32),
        pltpu.VMEM_SHARED((N,), jnp.float32),
        pltpu.VMEM_SHARED((N,), jnp.float32),
        pltpu.VMEM_SHARED((N,), jnp.float32),
        pltpu.VMEM_SHARED((N, H), jnp.float32),
        pltpu.SemaphoreType.DMA,
    ],
)
def _sc_gat0(src_hbm, dst_hbm, ew_hbm, asrc_hbm, adst_hbm, zt_hbm,
             den_out, degd_out, degs_out, acc_out,
             asrc_t, adst_t, src_v, dst_v, ew_v,
             srcbuf, dstbuf, ewbuf, exbuf, rows, zrows, zsc,
             den_sh, degd_sh, degs_sh, acc_sh, sem):
    cid = lax.axis_index("c")
    sid = lax.axis_index("s")
    wid = cid * NS + sid
    ebase = wid * EPT

    pltpu.sync_copy(src_hbm.at[pl.ds(ebase, EPT)], src_v)
    pltpu.sync_copy(dst_hbm.at[pl.ds(ebase, EPT)], dst_v)
    pltpu.sync_copy(ew_hbm.at[pl.ds(ebase, EPT)], ew_v)
    pltpu.sync_copy(asrc_hbm, asrc_t)
    pltpu.sync_copy(adst_hbm, adst_t)

    zv = jnp.zeros((L,), jnp.float32)
    for i in range(ZR):
        for q in range(H // L):
            zrows[i, pl.ds(L * q, L)] = zv
    for i in range(ZS // L):
        zsc[pl.ds(L * i, L)] = zv

    @pl.when(sid < NT)
    def _zero_accs():
        for k in range(SPAN // ZR):
            pltpu.sync_copy(zrows, acc_sh.at[pl.ds(sid * SPAN + k * ZR, ZR), :])
        pltpu.sync_copy(zsc, den_sh.at[pl.ds(sid * SPAN, ZS)])
        pltpu.sync_copy(zsc, degd_sh.at[pl.ds(sid * SPAN, ZS)])
        pltpu.sync_copy(zsc, degs_sh.at[pl.ds(sid * SPAN, ZS)])

    plsc.subcore_barrier()

    def chunk(j, carry):
        b = j * CH
        for t in range(CH // L):
            srcbuf[pl.ds(L * t, L)] = src_v[pl.ds(b + L * t, L)]
            dstbuf[pl.ds(L * t, L)] = dst_v[pl.ds(b + L * t, L)]
            ewbuf[pl.ds(L * t, L)] = ew_v[pl.ds(b + L * t, L)]
        pltpu.async_copy(zt_hbm.at[srcbuf], rows, sem).wait()
        for t in range(CH // L):
            si = srcbuf[pl.ds(L * t, L)]
            di = dstbuf[pl.ds(L * t, L)]
            lg = plsc.load_gather(asrc_t, [si]) + plsc.load_gather(adst_t, [di])
            lg = jnp.where(lg > 0, lg, 0.2 * lg)
            exbuf[pl.ds(L * t, L)] = jnp.exp(lg) * ewbuf[pl.ds(L * t, L)]
        for t in range(CH // L):
            ex16 = exbuf[pl.ds(L * t, L)]
            for k in range(L):
                i = L * t + k
                s = jnp.full((L,), ex16[k], jnp.float32)
                for q in range(H // L):
                    rows[i, pl.ds(L * q, L)] = rows[i, pl.ds(L * q, L)] * s
        pltpu.sync_copy(rows, acc_sh.at[dstbuf], add=True)
        pltpu.sync_copy(exbuf, den_sh.at[dstbuf], add=True)
        pltpu.sync_copy(ewbuf, degd_sh.at[dstbuf], add=True)
        pltpu.sync_copy(ewbuf, degs_sh.at[srcbuf], add=True)
        return carry

    lax.fori_loop(0, NCH, chunk, 0)
    plsc.subcore_barrier()

    @pl.when(sid < NT)
    def _copy_out():
        pltpu.sync_copy(acc_sh.at[pl.ds(sid * SPAN, SPAN), :],
                        acc_out.at[cid, pl.ds(sid * SPAN, SPAN), :])
        pltpu.sync_copy(den_sh.at[pl.ds(sid * SPAN, SPAN)],
                        den_out.at[cid, sid])
        pltpu.sync_copy(degd_sh.at[pl.ds(sid * SPAN, SPAN)],
                        degd_out.at[cid, sid])
        pltpu.sync_copy(degs_sh.at[pl.ds(sid * SPAN, SPAN)],
                        degs_out.at[cid, sid])


def _mid_body(a_ref, d_ref, dd_ref, ds_ref, bng_ref, bnb_ref, w1_ref, b1_ref,
              as1_ref, ad1_ref,
              zt1_ref, asrc1_ref, adst1_ref, rsds_ref, rsdd_ref):
    f32 = jnp.float32
    BN = a_ref.shape[1]
    den = d_ref[0, 0] + d_ref[1, 0] + 1e-16
    dent = den.reshape(BN, 1)
    hl1 = (a_ref[0] + a_ref[1]) / dent
    hl1 = jnp.maximum(hl1 * bng_ref[...] + bnb_ref[...], 0.0)
    zt1 = jnp.dot(hl1, w1_ref[...], preferred_element_type=f32)
    zt1 = zt1 + b1_ref[...]
    zt1_ref[...] = zt1
    asrc1_ref[...] = lax.dot_general(
        as1_ref[...], zt1, (((1,), (1,)), ((), ())),
        preferred_element_type=f32).reshape(1, 1, BN)
    adst1_ref[...] = lax.dot_general(
        ad1_ref[...], zt1, (((1,), (1,)), ((), ())),
        preferred_element_type=f32).reshape(1, 1, BN)
    rsds_ref[...] = lax.rsqrt(
        ds_ref[0, 0] + ds_ref[1, 0] + 1.0).reshape(1, 1, BN)
    rsdd_ref[...] = lax.rsqrt(
        dd_ref[0, 0] + dd_ref[1, 0] + 1.0).reshape(1, 1, BN)


def _mid(acc0, den0, degd, degs, p):
    BN = 2000
    NB = N // BN
    g1 = p["gat1"]
    g0 = p["gat0"]
    svec = lambda v: v.reshape(NC, NB, 1, BN)
    return pl.pallas_call(
        _mid_body,
        grid=(NB,),
        in_specs=[
            pl.BlockSpec((NC, BN, H), lambda i: (0, i, 0)),
            pl.BlockSpec((NC, 1, 1, BN), lambda i: (0, i, 0, 0)),
            pl.BlockSpec((NC, 1, 1, BN), lambda i: (0, i, 0, 0)),
            pl.BlockSpec((NC, 1, 1, BN), lambda i: (0, i, 0, 0)),
            pl.BlockSpec((1, H), lambda i: (0, 0)),
            pl.BlockSpec((1, H), lambda i: (0, 0)),
            pl.BlockSpec((H, H), lambda i: (0, 0)),
            pl.BlockSpec((1, H), lambda i: (0, 0)),
            pl.BlockSpec((1, H), lambda i: (0, 0)),
            pl.BlockSpec((1, H), lambda i: (0, 0)),
        ],
        out_specs=[
            pl.BlockSpec((BN, H), lambda i: (i, 0)),
            pl.BlockSpec((1, 1, BN), lambda i: (i, 0, 0)),
            pl.BlockSpec((1, 1, BN), lambda i: (i, 0, 0)),
            pl.BlockSpec((1, 1, BN), lambda i: (i, 0, 0)),
            pl.BlockSpec((1, 1, BN), lambda i: (i, 0, 0)),
        ],
        out_shape=[
            jax.ShapeDtypeStruct((N, H), jnp.float32),
            jax.ShapeDtypeStruct((NB, 1, BN), jnp.float32),
            jax.ShapeDtypeStruct((NB, 1, BN), jnp.float32),
            jax.ShapeDtypeStruct((NB, 1, BN), jnp.float32),
            jax.ShapeDtypeStruct((NB, 1, BN), jnp.float32),
        ],
    )(acc0, svec(den0), svec(degd), svec(degs),
      g0["bn_g"].reshape(1, H), g0["bn_b"].reshape(1, H),
      g1["W"], g1["b"].reshape(1, H),
      g1["a_src"].reshape(1, H), g1["a_dst"].reshape(1, H))


@functools.partial(
    pl.kernel,
    out_type=[
        jax.ShapeDtypeStruct((NC, NT, SPAN), jnp.float32),
        jax.ShapeDtypeStruct((NC, N, H), jnp.float32),
    ],
    mesh=_mesh,
    compiler_params=pltpu.CompilerParams(needs_layout_passes=False, use_tc_tiling_on_sc=False),
    scratch_types=[
        pltpu.VMEM((N,), jnp.float32),
        pltpu.VMEM((N,), jnp.float32),
        pltpu.VMEM((EPT,), jnp.int32),
        pltpu.VMEM((EPT,), jnp.int32),
        pltpu.VMEM((EPT,), jnp.float32),
        pltpu.VMEM((CH,), jnp.int32),
        pltpu.VMEM((CH,), jnp.int32),
        pltpu.VMEM((CH,), jnp.float32),
        pltpu.VMEM((CH,), jnp.float32),
        pltpu.VMEM((CH, H), jnp.float32),
        pltpu.VMEM((ZR, H), jnp.float32),
        pltpu.VMEM((ZS,), jnp.float32),
        pltpu.VMEM_SHARED((N,), jnp.float32),
        pltpu.VMEM_SHARED((N, H), jnp.float32),
        pltpu.SemaphoreType.DMA,
    ],
)
def _sc_gat1(src_hbm, dst_hbm, ew_hbm, asrc_hbm, adst_hbm, zt1_hbm,
             den_out, acc1_out,
             asrc_t, adst_t, src_v, dst_v, ew_v,
             srcbuf, dstbuf, ewbuf, exbuf, rows,
             zrows, zsc, den_sh, acc1_sh, sem):
    cid = lax.axis_index("c")
    sid = lax.axis_index("s")
    wid = cid * NS + sid
    ebase = wid * EPT

    pltpu.sync_copy(src_hbm.at[pl.ds(ebase, EPT)], src_v)
    pltpu.sync_copy(dst_hbm.at[pl.ds(ebase, EPT)], dst_v)
    pltpu.sync_copy(ew_hbm.at[pl.ds(ebase, EPT)], ew_v)
    pltpu.sync_copy(asrc_hbm, asrc_t)
    pltpu.sync_copy(adst_hbm, adst_t)

    zv = jnp.zeros((L,), jnp.float32)
    for i in range(ZR):
        for q in range(H // L):
            zrows[i, pl.ds(L * q, L)] = zv
    for i in range(ZS // L):
        zsc[pl.ds(L * i, L)] = zv

    @pl.when(sid < NT)
    def _zero_accs():
        for k in range(SPAN // ZR):
            pltpu.sync_copy(zrows, acc1_sh.at[pl.ds(sid * SPAN + k * ZR, ZR), :])
        pltpu.sync_copy(zsc, den_sh.at[pl.ds(sid * SPAN, ZS)])

    plsc.subcore_barrier()

    def chunk(j, carry):
        b = j * CH
        for t in range(CH // L):
            srcbuf[pl.ds(L * t, L)] = src_v[pl.ds(b + L * t, L)]
            dstbuf[pl.ds(L * t, L)] = dst_v[pl.ds(b + L * t, L)]
            ewbuf[pl.ds(L * t, L)] = ew_v[pl.ds(b + L * t, L)]
        cp1 = pltpu.async_copy(zt1_hbm.at[srcbuf], rows, sem)
        for t in range(CH // L):
            si = srcbuf[pl.ds(L * t, L)]
            di = dstbuf[pl.ds(L * t, L)]
            lg = plsc.load_gather(asrc_t, [si]) + plsc.load_gather(adst_t, [di])
            lg = jnp.where(lg > 0, lg, 0.2 * lg)
            exbuf[pl.ds(L * t, L)] = jnp.exp(lg) * ewbuf[pl.ds(L * t, L)]
        cp1.wait()
        for t in range(CH // L):
            ex16 = exbuf[pl.ds(L * t, L)]
            for k in range(L):
                i = L * t + k
                s = jnp.full((L,), ex16[k], jnp.float32)
                for q in range(H // L):
                    rows[i, pl.ds(L * q, L)] = rows[i, pl.ds(L * q, L)] * s
        pltpu.sync_copy(rows, acc1_sh.at[dstbuf], add=True)
        pltpu.sync_copy(exbuf, den_sh.at[dstbuf], add=True)
        return carry

    lax.fori_loop(0, NCH, chunk, 0)
    plsc.subcore_barrier()

    @pl.when(sid < NT)
    def _copy_out():
        pltpu.sync_copy(acc1_sh.at[pl.ds(sid * SPAN, SPAN), :],
                        acc1_out.at[cid, pl.ds(sid * SPAN, SPAN), :])
        pltpu.sync_copy(den_sh.at[pl.ds(sid * SPAN, SPAN)],
                        den_out.at[cid, sid])


@functools.partial(
    pl.kernel,
    out_type=[
        jax.ShapeDtypeStruct((NC, N, H), jnp.float32),
    ],
    mesh=_mesh,
    compiler_params=pltpu.CompilerParams(needs_layout_passes=False, use_tc_tiling_on_sc=False),
    scratch_types=[
        pltpu.VMEM((N,), jnp.float32),
        pltpu.VMEM((N,), jnp.float32),
        pltpu.VMEM((EPT,), jnp.int32),
        pltpu.VMEM((EPT,), jnp.int32),
        pltpu.VMEM((EPT,), jnp.float32),
        pltpu.VMEM((CH,), jnp.int32),
        pltpu.VMEM((CH,), jnp.int32),
        pltpu.VMEM((CH,), jnp.float32),
        pltpu.VMEM((CH, H), jnp.float32),
        pltpu.VMEM((ZR, H), jnp.float32),
        pltpu.VMEM_SHARED((N, H), jnp.float32),
        pltpu.SemaphoreType.DMA,
    ],
)
def _sc_gcn(src_hbm, dst_hbm, ew_hbm, rsds_hbm, rsdd_hbm, zz_hbm,
            accg_out,
            rsds_t, rsdd_t, src_v, dst_v, ew_v,
            srcbuf, dstbuf, nrbuf, rowsg, zrows, accg_sh, sem):
    cid = lax.axis_index("c")
    sid = lax.axis_index("s")
    wid = cid * NS + sid
    ebase = wid * EPT

    pltpu.sync_copy(src_hbm.at[pl.ds(ebase, EPT)], src_v)
    pltpu.sync_copy(dst_hbm.at[pl.ds(ebase, EPT)], dst_v)
    pltpu.sync_copy(ew_hbm.at[pl.ds(ebase, EPT)], ew_v)
    pltpu.sync_copy(rsds_hbm, rsds_t)
    pltpu.sync_copy(rsdd_hbm, rsdd_t)

    zv = jnp.zeros((L,), jnp.float32)
    for i in range(ZR):
        for q in range(H // L):
            zrows[i, pl.ds(L * q, L)] = zv

    @pl.when(sid < NT)
    def _zero_accs():
        for k in range(SPAN // ZR):
            pltpu.sync_copy(zrows, accg_sh.at[pl.ds(sid * SPAN + k * ZR, ZR), :])

    plsc.subcore_barrier()

    def chunk(j, carry):
        b = j * CH
        for t in range(CH // L):
            srcbuf[pl.ds(L * t, L)] = src_v[pl.ds(b + L * t, L)]
            dstbuf[pl.ds(L * t, L)] = dst_v[pl.ds(b + L * t, L)]
        cp = pltpu.async_copy(zz_hbm.at[srcbuf], rowsg, sem)
        for t in range(CH // L):
            si = srcbuf[pl.ds(L * t, L)]
            di = dstbuf[pl.ds(L * t, L)]
            nr = plsc.load_gather(rsds_t, [si]) * plsc.load_gather(rsdd_t, [di])
            nrbuf[pl.ds(L * t, L)] = nr * ew_v[pl.ds(b + L * t, L)]
        cp.wait()
        for t in range(CH // L):
            nr16 = nrbuf[pl.ds(L * t, L)]
            for k in range(L):
                i = L * t + k
                g = jnp.full((L,), nr16[k], jnp.float32)
                for q in range(H // L):
                    rowsg[i, pl.ds(L * q, L)] = rowsg[i, pl.ds(L * q, L)] * g
        pltpu.sync_copy(rowsg, accg_sh.at[dstbuf], add=True)
        return carry

    lax.fori_loop(0, NCH, chunk, 0)
    plsc.subcore_barrier()

    @pl.when(sid < NT)
    def _copy_out():
        pltpu.sync_copy(accg_sh.at[pl.ds(sid * SPAN, SPAN), :],
                        accg_out.at[cid, pl.ds(sid * SPAN, SPAN), :])


def _head_body(a_ref, d_ref, g_ref,
               bng_ref, bnb_ref, la1w_ref, la1b_ref, la2w_ref, la2b_ref,
               gcnb_ref, lng_ref, lnb_ref,
               ga1w_ref, ga1b_ref, ga2w_ref, ga2b_ref,
               gwl_ref, gwg_ref, gb_ref, f1l_ref, f1g_ref, f1b_ref,
               f2w_ref, f2b_ref, ow_ref, ob_ref,
               pred_ref, fused_ref):
    f32 = jnp.float32
    BN = a_ref.shape[1]
    den = d_ref[0, 0] + d_ref[1, 0] + 1e-16
    dent = den.reshape(BN, 1)
    hl2 = (a_ref[0] + a_ref[1]) / dent
    hl2 = jnp.maximum(hl2 * bng_ref[...] + bnb_ref[...], 0.0)
    t = jnp.maximum(jnp.dot(hl2, la1w_ref[...], preferred_element_type=f32)
                    + la1b_ref[...], 0.0)
    hl = hl2 + jnp.dot(t, la2w_ref[...], preferred_element_type=f32) \
        + la2b_ref[...]

    hg = g_ref[0] + g_ref[1] + gcnb_ref[...]
    mu = jnp.mean(hg, axis=-1, keepdims=True)
    var = jnp.mean((hg - mu) * (hg - mu), axis=-1, keepdims=True)
    hg = (hg - mu) * lax.rsqrt(var + 1e-5) * lng_ref[...] + lnb_ref[...]
    hg = jnp.maximum(hg, 0.0)
    t = jnp.maximum(jnp.dot(hg, ga1w_ref[...], preferred_element_type=f32)
                    + ga1b_ref[...], 0.0)
    hg = hg + jnp.dot(t, ga2w_ref[...], preferred_element_type=f32) \
        + ga2b_ref[...]

    glog = jnp.dot(hl, gwl_ref[...], preferred_element_type=f32) \
        + jnp.dot(hg, gwg_ref[...], preferred_element_type=f32) + gb_ref[...]
    gate = 1.0 / (1.0 + jnp.exp(-glog))
    t = jnp.maximum(jnp.dot(hl, f1l_ref[...], preferred_element_type=f32)
                    + jnp.dot(hg, f1g_ref[...], preferred_element_type=f32)
                    + f1b_ref[...], 0.0)
    adj = jnp.dot(t, f2w_ref[...], preferred_element_type=f32) + f2b_ref[...]
    fused = gate * hl + (1.0 - gate) * hg + adj
    fused_ref[...] = fused
    pred_ref[...] = (lax.dot_general(
        ow_ref[...], fused, (((1,), (1,)), ((), ())),
        preferred_element_type=f32) + ob_ref[...]).reshape(1, 1, BN)


def _head(acc1, den1, accg, p):
    BN = 2000
    NB = N // BN
    g1 = p["gat1"]
    gcn = p["gcn0"]
    row = lambda v: v.reshape(1, -1)
    bspec = [
        pl.BlockSpec((NC, BN, H), lambda i: (0, i, 0)),
        pl.BlockSpec((NC, 1, 1, BN), lambda i: (0, i, 0, 0)),
        pl.BlockSpec((NC, BN, H), lambda i: (0, i, 0)),
        pl.BlockSpec((1, H), lambda i: (0, 0)),
        pl.BlockSpec((1, H), lambda i: (0, 0)),
        pl.BlockSpec((H, A), lambda i: (0, 0)),
        pl.BlockSpec((1, A), lambda i: (0, 0)),
        pl.BlockSpec((A, H), lambda i: (0, 0)),
        pl.BlockSpec((1, H), lambda i: (0, 0)),
        pl.BlockSpec((1, H), lambda i: (0, 0)),
        pl.BlockSpec((1, H), lambda i: (0, 0)),
        pl.BlockSpec((1, H), lambda i: (0, 0)),
        pl.BlockSpec((H, A), lambda i: (0, 0)),
        pl.BlockSpec((1, A), lambda i: (0, 0)),
        pl.BlockSpec((A, H), lambda i: (0, 0)),
        pl.BlockSpec((1, H), lambda i: (0, 0)),
        pl.BlockSpec((H, H), lambda i: (0, 0)),
        pl.BlockSpec((H, H), lambda i: (0, 0)),
        pl.BlockSpec((1, H), lambda i: (0, 0)),
        pl.BlockSpec((H, H), lambda i: (0, 0)),
        pl.BlockSpec((H, H), lambda i: (0, 0)),
        pl.BlockSpec((1, H), lambda i: (0, 0)),
        pl.BlockSpec((H, H), lambda i: (0, 0)),
        pl.BlockSpec((1, H), lambda i: (0, 0)),
        pl.BlockSpec((1, H), lambda i: (0, 0)),
        pl.BlockSpec((1, 1), lambda i: (0, 0)),
    ]
    return pl.pallas_call(
        _head_body,
        grid=(N // BN,),
        in_specs=bspec,
        out_specs=[
            pl.BlockSpec((1, 1, BN), lambda i: (i, 0, 0)),
            pl.BlockSpec((BN, H), lambda i: (i, 0)),
        ],
        out_shape=[
            jax.ShapeDtypeStruct((NB, 1, BN), jnp.float32),
            jax.ShapeDtypeStruct((N, H), jnp.float32),
        ],
    )(acc1, den1.reshape(NC, NB, 1, BN), accg,
      row(g1["bn_g"]), row(g1["bn_b"]),
      p["la1_w"], row(p["la1_b"]), p["la2_w"], row(p["la2_b"]),
      row(gcn["b"]), row(gcn["ln_g"]), row(gcn["ln_b"]),
      p["ga1_w"], row(p["ga1_b"]), p["ga2_w"], row(p["ga2_b"]),
      p["gate_w"][:H], p["gate_w"][H:], row(p["gate_b"]),
      p["fa1_w"][:H], p["fa1_w"][H:], row(p["fa1_b"]),
      p["fa2_w"], row(p["fa2_b"]), p["out_w"].reshape(1, H),
      p["out_b"].reshape(1, 1))


def kernel(x_local, x_global, edge_attr, edge_index, params):
    src = edge_index[0]
    dst = edge_index[1]
    ew = _edge_weights(edge_attr.T, params).reshape(E)
    zt0, asrc0, adst0, zz = _node_prep(x_local, x_global, params)
    den0, degd, degs, acc0 = _sc_gat0(
        src, dst, ew, asrc0.reshape(N), adst0.reshape(N), zt0)
    den0 = den0.reshape(NC, N)
    degd = degd.reshape(NC, N)
    degs = degs.reshape(NC, N)
    acc0 = acc0.reshape(NC, N, H)
    zt1, asrc1, adst1, rsds, rsdd = _mid(acc0, den0, degd, degs, params)
    den1, acc1 = _sc_gat1(
        src, dst, ew, asrc1.reshape(N), adst1.reshape(N), zt1)
    (accg,) = _sc_gcn(
        src, dst, ew, rsds.reshape(N), rsdd.reshape(N), zz)
    den1 = den1.reshape(NC, N)
    acc1 = acc1.reshape(NC, N, H)
    accg = accg.reshape(NC, N, H)
    pred, fused = _head(acc1, den1, accg, params)
    return pred.reshape(N), fused


# speedup vs baseline: 36.1530x; 1.0122x over previous
import functools

import jax
import jax.numpy as jnp
from jax import lax
from jax.experimental import pallas as pl
from jax.experimental.pallas import tpu as pltpu
from jax.experimental.pallas import tpu_sc as plsc

N = 10000
E = 320000
D = 128
H = 64
A = 32
SIG2 = 900.0

NC = 2
NS = 16
NW = NC * NS
EPT = E // NW
CH = 80
NCH = EPT // CH
NT = 10
SPAN = N // NT
ZR = 200
ZS = 1000
L = 16

_mesh = plsc.VectorSubcoreMesh(core_axis_name="c", subcore_axis_name="s")


def _ew_body(ea_ref, w1_ref, b1_ref, w2_ref, b2_ref, out_ref):
    h = lax.dot_general(w1_ref[...], ea_ref[...], (((0,), (0,)), ((), ())),
                        preferred_element_type=jnp.float32)
    h = h + b1_ref[...]
    h = jnp.where(h > 0, h, jnp.exp(h) - 1.0)
    ind = jnp.dot(w2_ref[...], h, preferred_element_type=jnp.float32)
    ind = ind + b2_ref[...]
    BE = ind.shape[-1]
    out_ref[...] = jnp.exp(-(ind * ind) / SIG2).reshape(1, 1, BE)


def _edge_weights(ea_t, p):
    BE = 32000
    return pl.pallas_call(
        _ew_body,
        grid=(E // BE,),
        in_specs=[
            pl.BlockSpec((10, BE), lambda i: (0, i)),
            pl.BlockSpec((10, 10), lambda i: (0, 0)),
            pl.BlockSpec((10, 1), lambda i: (0, 0)),
            pl.BlockSpec((1, 10), lambda i: (0, 0)),
            pl.BlockSpec((1, 1), lambda i: (0, 0)),
        ],
        out_specs=pl.BlockSpec((1, 1, BE), lambda i: (i, 0, 0)),
        out_shape=jax.ShapeDtypeStruct((E // BE, 1, BE), jnp.float32),
    )(ea_t, p["ew1_w"], p["ew1_b"].reshape(10, 1),
      p["ew2_w"].reshape(1, 10), p["ew2_b"].reshape(1, 1))


def _prep_body(xl_ref, xg_ref, w0_ref, b0_ref, as_ref, ad_ref, wg_ref,
               zt_ref, asrc_ref, adst_ref, zz_ref):
    f32 = jnp.float32
    zt = jnp.dot(xl_ref[...], w0_ref[...], preferred_element_type=f32)
    zt = zt + b0_ref[...]
    zt_ref[...] = zt
    BN = zt.shape[0]
    asrc_ref[...] = lax.dot_general(
        as_ref[...], zt, (((1,), (1,)), ((), ())),
        preferred_element_type=f32).reshape(1, 1, BN)
    adst_ref[...] = lax.dot_general(
        ad_ref[...], zt, (((1,), (1,)), ((), ())),
        preferred_element_type=f32).reshape(1, 1, BN)
    zz_ref[...] = jnp.dot(xg_ref[...], wg_ref[...],
                          preferred_element_type=f32)


def _node_prep(x_local, x_global, p):
    BN = 2000
    NB = N // BN
    g0 = p["gat0"]
    return pl.pallas_call(
        _prep_body,
        grid=(NB,),
        in_specs=[
            pl.BlockSpec((BN, D), lambda i: (i, 0)),
            pl.BlockSpec((BN, D), lambda i: (i, 0)),
            pl.BlockSpec((D, H), lambda i: (0, 0)),
            pl.BlockSpec((1, H), lambda i: (0, 0)),
            pl.BlockSpec((1, H), lambda i: (0, 0)),
            pl.BlockSpec((1, H), lambda i: (0, 0)),
            pl.BlockSpec((D, H), lambda i: (0, 0)),
        ],
        out_specs=[
            pl.BlockSpec((BN, H), lambda i: (i, 0)),
            pl.BlockSpec((1, 1, BN), lambda i: (i, 0, 0)),
            pl.BlockSpec((1, 1, BN), lambda i: (i, 0, 0)),
            pl.BlockSpec((BN, H), lambda i: (i, 0)),
        ],
        out_shape=[
            jax.ShapeDtypeStruct((N, H), jnp.float32),
            jax.ShapeDtypeStruct((NB, 1, BN), jnp.float32),
            jax.ShapeDtypeStruct((NB, 1, BN), jnp.float32),
            jax.ShapeDtypeStruct((N, H), jnp.float32),
        ],
    )(x_local, x_global, g0["W"], g0["b"].reshape(1, H),
      g0["a_src"].reshape(1, H), g0["a_dst"].reshape(1, H), p["gcn0"]["W"])


@functools.partial(
    pl.kernel,
    out_type=[
        jax.ShapeDtypeStruct((NC, NT, SPAN), jnp.float32),
        jax.ShapeDtypeStruct((NC, NT, SPAN), jnp.float32),
        jax.ShapeDtypeStruct((NC, NT, SPAN), jnp.float32),
        jax.ShapeDtypeStruct((NC, N, H), jnp.float32),
    ],
    mesh=_mesh,
    compiler_params=pltpu.CompilerParams(needs_layout_passes=False, use_tc_tiling_on_sc=False),
    scratch_types=[
        pltpu.VMEM((N,), jnp.float32),
        pltpu.VMEM((N,), jnp.float32),
        pltpu.VMEM((EPT,), jnp.int32),
        pltpu.VMEM((EPT,), jnp.int32),
        pltpu.VMEM((EPT,), jnp.float32),
        pltpu.VMEM((CH,), jnp.int32),
        pltpu.VMEM((CH,), jnp.int32),
        pltpu.VMEM((CH,), jnp.float32),
        pltpu.VMEM((CH,), jnp.float32),
        pltpu.VMEM((CH, H), jnp.float32),
        pltpu.VMEM((ZR, H), jnp.float32),
        pltpu.VMEM((ZS,), jnp.float32),
        pltpu.VMEM_SHARED((N,), jnp.float32),
        pltpu.VMEM_SHARED((N,), jnp.float32),
        pltpu.VMEM_SHARED((N,), jnp.float32),
        pltpu.VMEM_SHARED((N, H), jnp.float32),
        pltpu.SemaphoreType.DMA,
    ],
)
def _sc_gat0(src_hbm, dst_hbm, ew_hbm, asrc_hbm, adst_hbm, zt_hbm,
             den_out, degd_out, degs_out, acc_out,
             asrc_t, adst_t, src_v, dst_v, ew_v,
             srcbuf, dstbuf, ewbuf, exbuf, rows, zrows, zsc,
             den_sh, degd_sh, degs_sh, acc_sh, sem):
    cid = lax.axis_index("c")
    sid = lax.axis_index("s")
    wid = cid * NS + sid
    ebase = wid * EPT

    pltpu.sync_copy(src_hbm.at[pl.ds(ebase, EPT)], src_v)
    pltpu.sync_copy(dst_hbm.at[pl.ds(ebase, EPT)], dst_v)
    pltpu.sync_copy(ew_hbm.at[pl.ds(ebase, EPT)], ew_v)
    pltpu.sync_copy(asrc_hbm, asrc_t)
    pltpu.sync_copy(adst_hbm, adst_t)

    zv = jnp.zeros((L,), jnp.float32)
    for i in range(ZR):
        for q in range(H // L):
            zrows[i, pl.ds(L * q, L)] = zv
    for i in range(ZS // L):
        zsc[pl.ds(L * i, L)] = zv

    @pl.when(sid < NT)
    def _zero_accs():
        for k in range(SPAN // ZR):
            pltpu.sync_copy(zrows, acc_sh.at[pl.ds(sid * SPAN + k * ZR, ZR), :])
        pltpu.sync_copy(zsc, den_sh.at[pl.ds(sid * SPAN, ZS)])
        pltpu.sync_copy(zsc, degd_sh.at[pl.ds(sid * SPAN, ZS)])
        pltpu.sync_copy(zsc, degs_sh.at[pl.ds(sid * SPAN, ZS)])

    plsc.subcore_barrier()

    def chunk(j, carry):
        b = j * CH
        for t in range(CH // L):
            srcbuf[pl.ds(L * t, L)] = src_v[pl.ds(b + L * t, L)]
            dstbuf[pl.ds(L * t, L)] = dst_v[pl.ds(b + L * t, L)]
            ewbuf[pl.ds(L * t, L)] = ew_v[pl.ds(b + L * t, L)]
        cp0 = pltpu.async_copy(zt_hbm.at[srcbuf], rows, sem)
        for t in range(CH // L):
            si = srcbuf[pl.ds(L * t, L)]
            di = dstbuf[pl.ds(L * t, L)]
            lg = plsc.load_gather(asrc_t, [si]) + plsc.load_gather(adst_t, [di])
            lg = jnp.where(lg > 0, lg, 0.2 * lg)
            exbuf[pl.ds(L * t, L)] = jnp.exp(lg) * ewbuf[pl.ds(L * t, L)]
        cp0.wait()
        for t in range(CH // L):
            ex16 = exbuf[pl.ds(L * t, L)]
            for k in range(L):
                i = L * t + k
                s = jnp.full((L,), ex16[k], jnp.float32)
                for q in range(H // L):
                    rows[i, pl.ds(L * q, L)] = rows[i, pl.ds(L * q, L)] * s
        pltpu.sync_copy(rows, acc_sh.at[dstbuf], add=True)
        pltpu.sync_copy(exbuf, den_sh.at[dstbuf], add=True)
        pltpu.sync_copy(ewbuf, degd_sh.at[dstbuf], add=True)
        pltpu.sync_copy(ewbuf, degs_sh.at[srcbuf], add=True)
        return carry

    lax.fori_loop(0, NCH, chunk, 0)
    plsc.subcore_barrier()

    @pl.when(sid < NT)
    def _copy_out():
        pltpu.sync_copy(acc_sh.at[pl.ds(sid * SPAN, SPAN), :],
                        acc_out.at[cid, pl.ds(sid * SPAN, SPAN), :])
        pltpu.sync_copy(den_sh.at[pl.ds(sid * SPAN, SPAN)],
                        den_out.at[cid, sid])
        pltpu.sync_copy(degd_sh.at[pl.ds(sid * SPAN, SPAN)],
                        degd_out.at[cid, sid])
        pltpu.sync_copy(degs_sh.at[pl.ds(sid * SPAN, SPAN)],
                        degs_out.at[cid, sid])


def _mid_body(a_ref, d_ref, dd_ref, ds_ref, bng_ref, bnb_ref, w1_ref, b1_ref,
              as1_ref, ad1_ref,
              zt1_ref, asrc1_ref, adst1_ref, rsds_ref, rsdd_ref):
    f32 = jnp.float32
    BN = a_ref.shape[1]
    den = d_ref[0, 0] + d_ref[1, 0] + 1e-16
    dent = den.reshape(BN, 1)
    hl1 = (a_ref[0] + a_ref[1]) / dent
    hl1 = jnp.maximum(hl1 * bng_ref[...] + bnb_ref[...], 0.0)
    zt1 = jnp.dot(hl1, w1_ref[...], preferred_element_type=f32)
    zt1 = zt1 + b1_ref[...]
    zt1_ref[...] = zt1
    asrc1_ref[...] = lax.dot_general(
        as1_ref[...], zt1, (((1,), (1,)), ((), ())),
        preferred_element_type=f32).reshape(1, 1, BN)
    adst1_ref[...] = lax.dot_general(
        ad1_ref[...], zt1, (((1,), (1,)), ((), ())),
        preferred_element_type=f32).reshape(1, 1, BN)
    rsds_ref[...] = lax.rsqrt(
        ds_ref[0, 0] + ds_ref[1, 0] + 1.0).reshape(1, 1, BN)
    rsdd_ref[...] = lax.rsqrt(
        dd_ref[0, 0] + dd_ref[1, 0] + 1.0).reshape(1, 1, BN)


def _mid(acc0, den0, degd, degs, p):
    BN = 2000
    NB = N // BN
    g1 = p["gat1"]
    g0 = p["gat0"]
    svec = lambda v: v.reshape(NC, NB, 1, BN)
    return pl.pallas_call(
        _mid_body,
        grid=(NB,),
        in_specs=[
            pl.BlockSpec((NC, BN, H), lambda i: (0, i, 0)),
            pl.BlockSpec((NC, 1, 1, BN), lambda i: (0, i, 0, 0)),
            pl.BlockSpec((NC, 1, 1, BN), lambda i: (0, i, 0, 0)),
            pl.BlockSpec((NC, 1, 1, BN), lambda i: (0, i, 0, 0)),
            pl.BlockSpec((1, H), lambda i: (0, 0)),
            pl.BlockSpec((1, H), lambda i: (0, 0)),
            pl.BlockSpec((H, H), lambda i: (0, 0)),
            pl.BlockSpec((1, H), lambda i: (0, 0)),
            pl.BlockSpec((1, H), lambda i: (0, 0)),
            pl.BlockSpec((1, H), lambda i: (0, 0)),
        ],
        out_specs=[
            pl.BlockSpec((BN, H), lambda i: (i, 0)),
            pl.BlockSpec((1, 1, BN), lambda i: (i, 0, 0)),
            pl.BlockSpec((1, 1, BN), lambda i: (i, 0, 0)),
            pl.BlockSpec((1, 1, BN), lambda i: (i, 0, 0)),
            pl.BlockSpec((1, 1, BN), lambda i: (i, 0, 0)),
        ],
        out_shape=[
            jax.ShapeDtypeStruct((N, H), jnp.float32),
            jax.ShapeDtypeStruct((NB, 1, BN), jnp.float32),
            jax.ShapeDtypeStruct((NB, 1, BN), jnp.float32),
            jax.ShapeDtypeStruct((NB, 1, BN), jnp.float32),
            jax.ShapeDtypeStruct((NB, 1, BN), jnp.float32),
        ],
    )(acc0, svec(den0), svec(degd), svec(degs),
      g0["bn_g"].reshape(1, H), g0["bn_b"].reshape(1, H),
      g1["W"], g1["b"].reshape(1, H),
      g1["a_src"].reshape(1, H), g1["a_dst"].reshape(1, H))


@functools.partial(
    pl.kernel,
    out_type=[
        jax.ShapeDtypeStruct((NC, NT, SPAN), jnp.float32),
        jax.ShapeDtypeStruct((NC, N, H), jnp.float32),
    ],
    mesh=_mesh,
    compiler_params=pltpu.CompilerParams(needs_layout_passes=False, use_tc_tiling_on_sc=False),
    scratch_types=[
        pltpu.VMEM((N,), jnp.float32),
        pltpu.VMEM((N,), jnp.float32),
        pltpu.VMEM((EPT,), jnp.int32),
        pltpu.VMEM((EPT,), jnp.int32),
        pltpu.VMEM((EPT,), jnp.float32),
        pltpu.VMEM((CH,), jnp.int32),
        pltpu.VMEM((CH,), jnp.int32),
        pltpu.VMEM((CH,), jnp.float32),
        pltpu.VMEM((CH,), jnp.float32),
        pltpu.VMEM((CH, H), jnp.float32),
        pltpu.VMEM((ZR, H), jnp.float32),
        pltpu.VMEM((ZS,), jnp.float32),
        pltpu.VMEM_SHARED((N,), jnp.float32),
        pltpu.VMEM_SHARED((N, H), jnp.float32),
        pltpu.SemaphoreType.DMA,
    ],
)
def _sc_gat1(src_hbm, dst_hbm, ew_hbm, asrc_hbm, adst_hbm, zt1_hbm,
             den_out, acc1_out,
             asrc_t, adst_t, src_v, dst_v, ew_v,
             srcbuf, dstbuf, ewbuf, exbuf, rows,
             zrows, zsc, den_sh, acc1_sh, sem):
    cid = lax.axis_index("c")
    sid = lax.axis_index("s")
    wid = cid * NS + sid
    ebase = wid * EPT

    pltpu.sync_copy(src_hbm.at[pl.ds(ebase, EPT)], src_v)
    pltpu.sync_copy(dst_hbm.at[pl.ds(ebase, EPT)], dst_v)
    pltpu.sync_copy(ew_hbm.at[pl.ds(ebase, EPT)], ew_v)
    pltpu.sync_copy(asrc_hbm, asrc_t)
    pltpu.sync_copy(adst_hbm, adst_t)

    zv = jnp.zeros((L,), jnp.float32)
    for i in range(ZR):
        for q in range(H // L):
            zrows[i, pl.ds(L * q, L)] = zv
    for i in range(ZS // L):
        zsc[pl.ds(L * i, L)] = zv

    @pl.when(sid < NT)
    def _zero_accs():
        for k in range(SPAN // ZR):
            pltpu.sync_copy(zrows, acc1_sh.at[pl.ds(sid * SPAN + k * ZR, ZR), :])
        pltpu.sync_copy(zsc, den_sh.at[pl.ds(sid * SPAN, ZS)])

    plsc.subcore_barrier()

    def chunk(j, carry):
        b = j * CH
        for t in range(CH // L):
            srcbuf[pl.ds(L * t, L)] = src_v[pl.ds(b + L * t, L)]
            dstbuf[pl.ds(L * t, L)] = dst_v[pl.ds(b + L * t, L)]
            ewbuf[pl.ds(L * t, L)] = ew_v[pl.ds(b + L * t, L)]
        cp1 = pltpu.async_copy(zt1_hbm.at[srcbuf], rows, sem)
        for t in range(CH // L):
            si = srcbuf[pl.ds(L * t, L)]
            di = dstbuf[pl.ds(L * t, L)]
            lg = plsc.load_gather(asrc_t, [si]) + plsc.load_gather(adst_t, [di])
            lg = jnp.where(lg > 0, lg, 0.2 * lg)
            exbuf[pl.ds(L * t, L)] = jnp.exp(lg) * ewbuf[pl.ds(L * t, L)]
        cp1.wait()
        for t in range(CH // L):
            ex16 = exbuf[pl.ds(L * t, L)]
            for k in range(L):
                i = L * t + k
                s = jnp.full((L,), ex16[k], jnp.float32)
                for q in range(H // L):
                    rows[i, pl.ds(L * q, L)] = rows[i, pl.ds(L * q, L)] * s
        pltpu.sync_copy(rows, acc1_sh.at[dstbuf], add=True)
        pltpu.sync_copy(exbuf, den_sh.at[dstbuf], add=True)
        return carry

    lax.fori_loop(0, NCH, chunk, 0)
    plsc.subcore_barrier()

    @pl.when(sid < NT)
    def _copy_out():
        pltpu.sync_copy(acc1_sh.at[pl.ds(sid * SPAN, SPAN), :],
                        acc1_out.at[cid, pl.ds(sid * SPAN, SPAN), :])
        pltpu.sync_copy(den_sh.at[pl.ds(sid * SPAN, SPAN)],
                        den_out.at[cid, sid])


@functools.partial(
    pl.kernel,
    out_type=[
        jax.ShapeDtypeStruct((NC, N, H), jnp.float32),
    ],
    mesh=_mesh,
    compiler_params=pltpu.CompilerParams(needs_layout_passes=False, use_tc_tiling_on_sc=False),
    scratch_types=[
        pltpu.VMEM((N,), jnp.float32),
        pltpu.VMEM((N,), jnp.float32),
        pltpu.VMEM((EPT,), jnp.int32),
        pltpu.VMEM((EPT,), jnp.int32),
        pltpu.VMEM((EPT,), jnp.float32),
        pltpu.VMEM((CH,), jnp.int32),
        pltpu.VMEM((CH,), jnp.int32),
        pltpu.VMEM((CH,), jnp.float32),
        pltpu.VMEM((CH, H), jnp.float32),
        pltpu.VMEM((ZR, H), jnp.float32),
        pltpu.VMEM_SHARED((N, H), jnp.float32),
        pltpu.SemaphoreType.DMA,
    ],
)
def _sc_gcn(src_hbm, dst_hbm, ew_hbm, rsds_hbm, rsdd_hbm, zz_hbm,
            accg_out,
            rsds_t, rsdd_t, src_v, dst_v, ew_v,
            srcbuf, dstbuf, nrbuf, rowsg, zrows, accg_sh, sem):
    cid = lax.axis_index("c")
    sid = lax.axis_index("s")
    wid = cid * NS + sid
    ebase = wid * EPT

    pltpu.sync_copy(src_hbm.at[pl.ds(ebase, EPT)], src_v)
    pltpu.sync_copy(dst_hbm.at[pl.ds(ebase, EPT)], dst_v)
    pltpu.sync_copy(ew_hbm.at[pl.ds(ebase, EPT)], ew_v)
    pltpu.sync_copy(rsds_hbm, rsds_t)
    pltpu.sync_copy(rsdd_hbm, rsdd_t)

    zv = jnp.zeros((L,), jnp.float32)
    for i in range(ZR):
        for q in range(H // L):
            zrows[i, pl.ds(L * q, L)] = zv

    @pl.when(sid < NT)
    def _zero_accs():
        for k in range(SPAN // ZR):
            pltpu.sync_copy(zrows, accg_sh.at[pl.ds(sid * SPAN + k * ZR, ZR), :])

    plsc.subcore_barrier()

    def chunk(j, carry):
        b = j * CH
        for t in range(CH // L):
            srcbuf[pl.ds(L * t, L)] = src_v[pl.ds(b + L * t, L)]
            dstbuf[pl.ds(L * t, L)] = dst_v[pl.ds(b + L * t, L)]
        cp = pltpu.async_copy(zz_hbm.at[srcbuf], rowsg, sem)
        for t in range(CH // L):
            si = srcbuf[pl.ds(L * t, L)]
            di = dstbuf[pl.ds(L * t, L)]
            nr = plsc.load_gather(rsds_t, [si]) * plsc.load_gather(rsdd_t, [di])
            nrbuf[pl.ds(L * t, L)] = nr * ew_v[pl.ds(b + L * t, L)]
        cp.wait()
        for t in range(CH // L):
            nr16 = nrbuf[pl.ds(L * t, L)]
            for k in range(L):
                i = L * t + k
                g = jnp.full((L,), nr16[k], jnp.float32)
                for q in range(H // L):
                    rowsg[i, pl.ds(L * q, L)] = rowsg[i, pl.ds(L * q, L)] * g
        pltpu.sync_copy(rowsg, accg_sh.at[dstbuf], add=True)
        return carry

    lax.fori_loop(0, NCH, chunk, 0)
    plsc.subcore_barrier()

    @pl.when(sid < NT)
    def _copy_out():
        pltpu.sync_copy(accg_sh.at[pl.ds(sid * SPAN, SPAN), :],
                        accg_out.at[cid, pl.ds(sid * SPAN, SPAN), :])


def _head_body(a_ref, d_ref, g_ref,
               bng_ref, bnb_ref, la1w_ref, la1b_ref, la2w_ref, la2b_ref,
               gcnb_ref, lng_ref, lnb_ref,
               ga1w_ref, ga1b_ref, ga2w_ref, ga2b_ref,
               gwl_ref, gwg_ref, gb_ref, f1l_ref, f1g_ref, f1b_ref,
               f2w_ref, f2b_ref, ow_ref, ob_ref,
               pred_ref, fused_ref):
    f32 = jnp.float32
    BN = a_ref.shape[1]
    den = d_ref[0, 0] + d_ref[1, 0] + 1e-16
    dent = den.reshape(BN, 1)
    hl2 = (a_ref[0] + a_ref[1]) / dent
    hl2 = jnp.maximum(hl2 * bng_ref[...] + bnb_ref[...], 0.0)
    t = jnp.maximum(jnp.dot(hl2, la1w_ref[...], preferred_element_type=f32)
                    + la1b_ref[...], 0.0)
    hl = hl2 + jnp.dot(t, la2w_ref[...], preferred_element_type=f32) \
        + la2b_ref[...]

    hg = g_ref[0] + g_ref[1] + gcnb_ref[...]
    mu = jnp.mean(hg, axis=-1, keepdims=True)
    var = jnp.mean((hg - mu) * (hg - mu), axis=-1, keepdims=True)
    hg = (hg - mu) * lax.rsqrt(var + 1e-5) * lng_ref[...] + lnb_ref[...]
    hg = jnp.maximum(hg, 0.0)
    t = jnp.maximum(jnp.dot(hg, ga1w_ref[...], preferred_element_type=f32)
                    + ga1b_ref[...], 0.0)
    hg = hg + jnp.dot(t, ga2w_ref[...], preferred_element_type=f32) \
        + ga2b_ref[...]

    glog = jnp.dot(hl, gwl_ref[...], preferred_element_type=f32) \
        + jnp.dot(hg, gwg_ref[...], preferred_element_type=f32) + gb_ref[...]
    gate = 1.0 / (1.0 + jnp.exp(-glog))
    t = jnp.maximum(jnp.dot(hl, f1l_ref[...], preferred_element_type=f32)
                    + jnp.dot(hg, f1g_ref[...], preferred_element_type=f32)
                    + f1b_ref[...], 0.0)
    adj = jnp.dot(t, f2w_ref[...], preferred_element_type=f32) + f2b_ref[...]
    fused = gate * hl + (1.0 - gate) * hg + adj
    fused_ref[...] = fused
    pred_ref[...] = (lax.dot_general(
        ow_ref[...], fused, (((1,), (1,)), ((), ())),
        preferred_element_type=f32) + ob_ref[...]).reshape(1, 1, BN)


def _head(acc1, den1, accg, p):
    BN = 2000
    NB = N // BN
    g1 = p["gat1"]
    gcn = p["gcn0"]
    row = lambda v: v.reshape(1, -1)
    bspec = [
        pl.BlockSpec((NC, BN, H), lambda i: (0, i, 0)),
        pl.BlockSpec((NC, 1, 1, BN), lambda i: (0, i, 0, 0)),
        pl.BlockSpec((NC, BN, H), lambda i: (0, i, 0)),
        pl.BlockSpec((1, H), lambda i: (0, 0)),
        pl.BlockSpec((1, H), lambda i: (0, 0)),
        pl.BlockSpec((H, A), lambda i: (0, 0)),
        pl.BlockSpec((1, A), lambda i: (0, 0)),
        pl.BlockSpec((A, H), lambda i: (0, 0)),
        pl.BlockSpec((1, H), lambda i: (0, 0)),
        pl.BlockSpec((1, H), lambda i: (0, 0)),
        pl.BlockSpec((1, H), lambda i: (0, 0)),
        pl.BlockSpec((1, H), lambda i: (0, 0)),
        pl.BlockSpec((H, A), lambda i: (0, 0)),
        pl.BlockSpec((1, A), lambda i: (0, 0)),
        pl.BlockSpec((A, H), lambda i: (0, 0)),
        pl.BlockSpec((1, H), lambda i: (0, 0)),
        pl.BlockSpec((H, H), lambda i: (0, 0)),
        pl.BlockSpec((H, H), lambda i: (0, 0)),
        pl.BlockSpec((1, H), lambda i: (0, 0)),
        pl.BlockSpec((H, H), lambda i: (0, 0)),
        pl.BlockSpec((H, H), lambda i: (0, 0)),
        pl.BlockSpec((1, H), lambda i: (0, 0)),
        pl.BlockSpec((H, H), lambda i: (0, 0)),
        pl.BlockSpec((1, H), lambda i: (0, 0)),
        pl.BlockSpec((1, H), lambda i: (0, 0)),
        pl.BlockSpec((1, 1), lambda i: (0, 0)),
    ]
    return pl.pallas_call(
        _head_body,
        grid=(N // BN,),
        in_specs=bspec,
        out_specs=[
            pl.BlockSpec((1, 1, BN), lambda i: (i, 0, 0)),
            pl.BlockSpec((BN, H), lambda i: (i, 0)),
        ],
        out_shape=[
            jax.ShapeDtypeStruct((NB, 1, BN), jnp.float32),
            jax.ShapeDtypeStruct((N, H), jnp.float32),
        ],
    )(acc1, den1.reshape(NC, NB, 1, BN), accg,
      row(g1["bn_g"]), row(g1["bn_b"]),
      p["la1_w"], row(p["la1_b"]), p["la2_w"], row(p["la2_b"]),
      row(gcn["b"]), row(gcn["ln_g"]), row(gcn["ln_b"]),
      p["ga1_w"], row(p["ga1_b"]), p["ga2_w"], row(p["ga2_b"]),
      p["gate_w"][:H], p["gate_w"][H:], row(p["gate_b"]),
      p["fa1_w"][:H], p["fa1_w"][H:], row(p["fa1_b"]),
      p["fa2_w"], row(p["fa2_b"]), p["out_w"].reshape(1, H),
      p["out_b"].reshape(1, 1))


def kernel(x_local, x_global, edge_attr, edge_index, params):
    src = edge_index[0]
    dst = edge_index[1]
    ew = _edge_weights(edge_attr.T, params).reshape(E)
    zt0, asrc0, adst0, zz = _node_prep(x_local, x_global, params)
    den0, degd, degs, acc0 = _sc_gat0(
        src, dst, ew, asrc0.reshape(N), adst0.reshape(N), zt0)
    den0 = den0.reshape(NC, N)
    degd = degd.reshape(NC, N)
    degs = degs.reshape(NC, N)
    acc0 = acc0.reshape(NC, N, H)
    zt1, asrc1, adst1, rsds, rsdd = _mid(acc0, den0, degd, degs, params)
    den1, acc1 = _sc_gat1(
        src, dst, ew, asrc1.reshape(N), adst1.reshape(N), zt1)
    (accg,) = _sc_gcn(
        src, dst, ew, rsds.reshape(N), rsdd.reshape(N), zz)
    den1 = den1.reshape(NC, N)
    acc1 = acc1.reshape(NC, N, H)
    accg = accg.reshape(NC, N, H)
    pred, fused = _head(acc1, den1, accg, params)
    return pred.reshape(N), fused
